# baseline (speedup 1.0000x reference)
import functools

import jax
import jax.numpy as jnp
from jax.experimental import pallas as pl
from jax.experimental.pallas import tpu as pltpu

_D = 256
_INV_SQRT_D = 1.0 / 16.0


def _fused_kernel(wp_ref, x_ref, wl_ref, buffer_ref, mem_ref, wq_ref, wo_ref,
                  out_ref, xs_ref, score_ref, pooled_ref, *, nsteps, blk,
                  total):
    p = pl.program_id(0)
    i = pl.program_id(1)

    @pl.when(p == 0)
    def _phase0():
        x = x_ref[...]
        xs_ref[pl.ds(i * blk, blk), :] = x
        d0 = jnp.dot(x, wl_ref[...].T, preferred_element_type=jnp.float32)
        s = jnp.sum(jnp.abs(d0)).reshape(1, 1)
        pv = jnp.sum(x, axis=0, keepdims=True)

        @pl.when(i == 0)
        def _():
            score_ref[...] = s
            pooled_ref[...] = pv

        @pl.when(i > 0)
        def _():
            score_ref[...] += s
            pooled_ref[...] += pv

    @pl.when(p == 1)
    def _phase1():
        inv = 1.0 / total
        wp = wp_ref[0]
        row = jax.lax.broadcasted_iota(jnp.int32, (5, 1), 0)
        buf = jnp.where(row == wp, pooled_ref[...] * inv, buffer_ref[...])

        score = score_ref[...] * inv
        fire = jax.nn.sigmoid(score - 2.0)
        temp = jnp.maximum(0.5, 1.0 - 0.3 * fire)

        x = xs_ref[pl.ds(i * blk, blk), :]
        logits = jnp.dot(x, buf.T, preferred_element_type=jnp.float32)
        logits = logits * _INV_SQRT_D
        m = jnp.max(logits, axis=-1, keepdims=True)
        e = jnp.exp(logits - m)
        attn = e / jnp.sum(e, axis=-1, keepdims=True)
        filtered = x + jnp.dot(attn, buf, preferred_element_type=jnp.float32)

        q = jnp.dot(filtered, wq_ref[...], preferred_element_type=jnp.float32)
        wl = jnp.dot(q, mem_ref[...].T, preferred_element_type=jnp.float32)
        wl = wl * (_INV_SQRT_D / temp)
        m2 = jnp.max(wl, axis=-1, keepdims=True)
        e2 = jnp.exp(wl - m2)
        wa = e2 / jnp.sum(e2, axis=-1, keepdims=True)
        read = jnp.dot(wa, mem_ref[...], preferred_element_type=jnp.float32)
        out_ref[...] = jnp.dot(read, wo_ref[...],
                               preferred_element_type=jnp.float32) + filtered


@jax.jit
def kernel(sensory_input, context, buffer, w_light, mem, W_q, W_o, write_ptr):
    B, S, d = sensory_input.shape
    total = B * S
    x = sensory_input.reshape(total, d)
    wp = jnp.asarray(write_ptr, dtype=jnp.int32).reshape(1)

    blk = 2048
    n = total // blk
    out = pl.pallas_call(
        functools.partial(_fused_kernel, nsteps=n, blk=blk, total=float(total)),
        grid=(2, n),
        in_specs=[
            pl.BlockSpec(memory_space=pltpu.SMEM),
            pl.BlockSpec((blk, d), lambda p, i: (jnp.where(p == 0, i, 0), 0)),
            pl.BlockSpec((1, d), lambda p, i: (0, 0)),
            pl.BlockSpec((5, d), lambda p, i: (0, 0)),
            pl.BlockSpec((7, d), lambda p, i: (0, 0)),
            pl.BlockSpec((d, d), lambda p, i: (0, 0)),
            pl.BlockSpec((d, d), lambda p, i: (0, 0)),
        ],
        out_specs=pl.BlockSpec((blk, d), lambda p, i: (jnp.where(p == 0, 0, i), 0)),
        out_shape=jax.ShapeDtypeStruct((total, d), jnp.float32),
        scratch_shapes=[
            pltpu.VMEM((total, d), jnp.float32),
            pltpu.VMEM((1, 1), jnp.float32),
            pltpu.VMEM((1, d), jnp.float32),
        ],
    )(wp, x, w_light.reshape(1, d), buffer, mem, W_q, W_o)

    return out.reshape(B, S, d)
